# Initial kernel scaffold; baseline (speedup 1.0000x reference)
import jax
import jax.numpy as jnp
from jax.experimental import pallas as pl


def kernel(seq, adj, W, bias):
    raise NotImplementedError("write your pallas kernel here")


# speedup vs baseline: 1.0286x; 1.0286x over previous
import jax
import jax.numpy as jnp
from jax.experimental import pallas as pl
from jax.experimental.pallas import tpu as pltpu

_BLK = 400


def _gcn_block_kernel(seq_ref, wt_ref, bias_ref, adj_ref, out_ref, fts_ref):
    @pl.when(pl.program_id(0) == 0)
    def _project():
        fts_ref[...] = jnp.dot(
            seq_ref[...], wt_ref[...], preferred_element_type=jnp.float32
        ).astype(jnp.bfloat16)

    acc = jnp.dot(
        adj_ref[...].astype(jnp.bfloat16),
        fts_ref[...],
        preferred_element_type=jnp.float32,
    )
    out_ref[...] = acc + bias_ref[...]


@jax.jit
def kernel(seq, adj, W, bias):
    b, n, d_in = seq.shape
    d_out = W.shape[0]
    seq2 = seq.reshape(n, d_in)
    adj2 = adj.reshape(n, n)
    wt = W.T
    bias2 = bias.reshape(1, d_out)

    out = pl.pallas_call(
        _gcn_block_kernel,
        grid=(n // _BLK,),
        in_specs=[
            pl.BlockSpec((n, d_in), lambda i: (0, 0)),
            pl.BlockSpec((d_in, d_out), lambda i: (0, 0)),
            pl.BlockSpec((1, d_out), lambda i: (0, 0)),
            pl.BlockSpec((_BLK, n), lambda i: (i, 0)),
        ],
        out_specs=pl.BlockSpec((_BLK, d_out), lambda i: (i, 0)),
        out_shape=jax.ShapeDtypeStruct((n, d_out), jnp.float32),
        scratch_shapes=[pltpu.VMEM((n, d_out), jnp.bfloat16)],
    )(seq2, wt, bias2, adj2)
    return out.reshape(b, n, d_out)
